# baseline (speedup 1.0000x reference)
import functools
import math

import jax
import jax.numpy as jnp
from jax import lax
from jax.experimental import pallas as pl
from jax.experimental.pallas import tpu as pltpu
from jax.experimental.pallas import tpu_sc as plsc

N = 10000
E = 320000
G = 64

CHUNK = 128
CPT = 160
EPW = CPT * CHUNK
EPAD = 16 * EPW
NROWS = 10240
RPT = NROWS // 16
NB = 5

BN_SCALE = float(1.0 / math.sqrt(1.0 + 1e-5))


def _sc_seg_body(h2_hbm, srcp2_hbm, dstp_hbm, agg_out,
                 src_v, dst_v, rows, agg_sh, gsem, ssem):
  c = lax.axis_index("c")
  s = lax.axis_index("s")

  zbuf = rows[0]

  def _zrow(i, _):
    for k in range(4):
      zbuf[i, pl.ds(k * 16, 16)] = jnp.zeros((16,), jnp.float32)
    return 0

  lax.fori_loop(0, CHUNK, _zrow, 0)
  pltpu.async_copy(srcp2_hbm.at[c, pl.ds(s * CPT, CPT)], src_v, gsem.at[0])
  pltpu.async_copy(dstp_hbm.at[pl.ds(s * CPT, CPT)], dst_v, gsem.at[1])
  for k in range(RPT // CHUNK):
    pltpu.async_copy(zbuf, agg_sh.at[pl.ds(s * RPT + k * CHUNK, CHUNK)],
                     ssem.at[k % NB])
  pltpu.make_async_copy(srcp2_hbm.at[c, pl.ds(s * CPT, CPT)], src_v,
                        gsem.at[0]).wait()
  pltpu.make_async_copy(dstp_hbm.at[pl.ds(s * CPT, CPT)], dst_v,
                        gsem.at[1]).wait()
  for k in range(RPT // CHUNK):
    pltpu.make_async_copy(zbuf, agg_sh.at[pl.ds(s * RPT + k * CHUNK, CHUNK)],
                          ssem.at[k % NB]).wait()
  plsc.subcore_barrier()

  for b in range(NB):
    pltpu.async_copy(h2_hbm.at[src_v.at[b]], rows[b], gsem.at[b])

  def _group(g, _):
    for b in range(NB):
      j = g * NB + b
      pltpu.make_async_copy(h2_hbm.at[src_v.at[j]], rows[b],
                            gsem.at[b]).wait()
      pltpu.async_copy(rows[b], agg_sh.at[dst_v.at[j]], ssem.at[b], add=True)

      @pl.when(j + NB < CPT)
      def _():
        pltpu.make_async_copy(rows[b], agg_sh.at[dst_v.at[j]],
                              ssem.at[b]).wait()
        pltpu.async_copy(h2_hbm.at[src_v.at[j + NB]], rows[b], gsem.at[b])

    return 0

  lax.fori_loop(0, CPT // NB, _group, 0)
  for b in range(NB):
    j = CPT - NB + b
    pltpu.make_async_copy(rows[b], agg_sh.at[dst_v.at[j]], ssem.at[b]).wait()

  plsc.subcore_barrier()

  KOUT = RPT // CHUNK

  def _row(k):
    return s * RPT + k * CHUNK

  for k in range(min(NB, KOUT)):
    pltpu.async_copy(agg_sh.at[pl.ds(_row(k), CHUNK)], rows[k], gsem.at[k])
  for k in range(KOUT):
    b = k % NB
    pltpu.make_async_copy(agg_sh.at[pl.ds(_row(k), CHUNK)], rows[b],
                          gsem.at[b]).wait()
    pltpu.async_copy(rows[b], agg_out.at[c, pl.ds(_row(k), CHUNK)],
                     ssem.at[b])
    if k + NB < KOUT:
      pltpu.make_async_copy(rows[b], agg_out.at[c, pl.ds(_row(k), CHUNK)],
                            ssem.at[b]).wait()
      pltpu.async_copy(agg_sh.at[pl.ds(_row(k + NB), CHUNK)], rows[b],
                       gsem.at[b])
  for k in range(max(KOUT - NB, 0), KOUT):
    b = k % NB
    pltpu.make_async_copy(rows[b], agg_out.at[c, pl.ds(_row(k), CHUNK)],
                          ssem.at[b]).wait()


def _make_sc_seg():
  mesh = plsc.VectorSubcoreMesh(core_axis_name="c", subcore_axis_name="s")
  out_type = jax.ShapeDtypeStruct((2, NROWS, 64), jnp.float32)
  scratch = [
      pltpu.VMEM((CPT, CHUNK), jnp.int32),
      pltpu.VMEM((CPT, CHUNK), jnp.int32),
      [pltpu.VMEM((CHUNK, 64), jnp.float32) for _ in range(NB)],
      pltpu.VMEM_SHARED((NROWS, 64), jnp.float32),
      pltpu.SemaphoreType.DMA((NB,)),
      pltpu.SemaphoreType.DMA((NB,)),
  ]
  return pl.kernel(_sc_seg_body,
                   out_type=out_type, mesh=mesh, scratch_types=scratch,
                   compiler_params=pltpu.CompilerParams(
                       use_tc_tiling_on_sc=False),
                   name="sc_seg")


_sc_seg = _make_sc_seg()

CCPT = 80


def _sc_cnt_body(dstp_hbm, cnt_out, dst_v, ones_v, stage_v, cnt_sh, ssem):
  c = lax.axis_index("c")
  s = lax.axis_index("s")
  wid = c * 16 + s

  def _orow(i, _):
    ones_v[i, :] = jnp.ones((16,), jnp.float32)
    return 0

  lax.fori_loop(0, CHUNK, _orow, 0)

  def _srow(i, _):
    stage_v[i, :] = jnp.zeros((16,), jnp.float32)
    return 0

  lax.fori_loop(0, RPT, _srow, 0)
  pltpu.sync_copy(stage_v, cnt_sh.at[pl.ds(s * RPT, RPT)])
  plsc.subcore_barrier()

  pltpu.sync_copy(dstp_hbm.at[pl.ds(wid * CCPT, CCPT)], dst_v)

  def _grp(g, _):
    for b in range(NB):
      j = g * NB + b
      pltpu.async_copy(ones_v, cnt_sh.at[dst_v.at[j]], ssem.at[b], add=True)
    for b in range(NB):
      j = g * NB + b
      pltpu.make_async_copy(ones_v, cnt_sh.at[dst_v.at[j]],
                            ssem.at[b]).wait()
    return 0

  lax.fori_loop(0, CCPT // NB, _grp, 0)
  plsc.subcore_barrier()

  pltpu.sync_copy(cnt_sh.at[pl.ds(s * RPT, RPT)], stage_v)
  pltpu.sync_copy(stage_v, cnt_out.at[c, pl.ds(s * RPT, RPT)])


def _make_sc_cnt():
  mesh = plsc.VectorSubcoreMesh(core_axis_name="c", subcore_axis_name="s")
  out_type = jax.ShapeDtypeStruct((2, NROWS, 16), jnp.float32)
  scratch = [
      pltpu.VMEM((CCPT, CHUNK), jnp.int32),
      pltpu.VMEM((CHUNK, 16), jnp.float32),
      pltpu.VMEM((RPT, 16), jnp.float32),
      pltpu.VMEM_SHARED((NROWS, 16), jnp.float32),
      pltpu.SemaphoreType.DMA((NB,)),
  ]
  return pl.kernel(_sc_cnt_body,
                   out_type=out_type, mesh=mesh, scratch_types=scratch,
                   compiler_params=pltpu.CompilerParams(
                       use_tc_tiling_on_sc=False),
                   name="sc_cnt")


_sc_cnt = _make_sc_cnt()


BT = 1000


def _tc_layer_body(aggP_ref, cnt_ref, h_ref, Wl_ref, Wr_ref, bl_ref,
                   g_ref, be_ref, out_ref):
  agg = jnp.concatenate([aggP_ref[0], aggP_ref[1]], axis=1)
  cnt = cnt_ref[0, :, 0:1] + cnt_ref[1, :, 0:1]
  denom = jnp.maximum(cnt, 1.0)
  agg = agg / denom
  h = h_ref[...]
  z = (jnp.dot(agg, Wl_ref[...], preferred_element_type=jnp.float32)
       + bl_ref[...][None, :]
       + jnp.dot(h, Wr_ref[...], preferred_element_type=jnp.float32))
  z = g_ref[...][None, :] * (z * BN_SCALE) + be_ref[...][None, :]
  out_ref[...] = jnp.maximum(z, 0.0)


def _tc_layer(aggP, cnt, h, Wl, Wr, bl, g, be):
  grid = N // BT
  return pl.pallas_call(
      _tc_layer_body,
      grid=(grid,),
      in_specs=[
          pl.BlockSpec((2, BT, 64), lambda i: (0, i, 0)),
          pl.BlockSpec((2, BT, 16), lambda i: (0, i, 0)),
          pl.BlockSpec((BT, 128), lambda i: (i, 0)),
          pl.BlockSpec((128, 128), lambda i: (0, 0)),
          pl.BlockSpec((128, 128), lambda i: (0, 0)),
          pl.BlockSpec((128,), lambda i: (0,)),
          pl.BlockSpec((128,), lambda i: (0,)),
          pl.BlockSpec((128,), lambda i: (0,)),
      ],
      out_specs=pl.BlockSpec((BT, 128), lambda i: (i, 0)),
      out_shape=jax.ShapeDtypeStruct((N, 128), jnp.float32),
  )(aggP, cnt, h, Wl, Wr, bl, g, be)


def _tc_final_body(f1_ref, f2_ref, f3_ref, xc_ref, batch_ref, other_ref,
                   W1_ref, b1_ref, W2_ref, b2_ref, W3_ref, b3_ref,
                   Rw1_ref, Rb1_ref, Rw2_ref, Rb2_ref,
                   no_ref, reg_ref, accJ_ref, accD_ref, accC_ref):
  i = pl.program_id(0)
  jk = jnp.concatenate([f1_ref[...], f2_ref[...], f3_ref[...]], axis=1)
  t = jnp.maximum(jnp.dot(jk, W1_ref[...],
                          preferred_element_type=jnp.float32)
                  + b1_ref[...][None, :], 0.0)
  t = jnp.maximum(jnp.dot(t, W2_ref[...],
                          preferred_element_type=jnp.float32)
                  + b2_ref[...][None, :], 0.0)
  no = jnp.dot(t, W3_ref[...], preferred_element_type=jnp.float32) \
      + b3_ref[...]
  no_ref[...] = no
  delay = no * xc_ref[...]

  onehot = (lax.broadcasted_iota(jnp.int32, (G, BT), 0)
            == batch_ref[0, 0][None, :]).astype(jnp.float32)

  @pl.when(i == 0)
  def _():
    accJ_ref[...] = jnp.zeros_like(accJ_ref)
    accD_ref[...] = jnp.zeros_like(accD_ref)
    accC_ref[...] = jnp.zeros_like(accC_ref)

  accJ_ref[...] += jnp.dot(onehot, jk, preferred_element_type=jnp.float32)
  accD_ref[...] += jnp.dot(onehot, delay, preferred_element_type=jnp.float32)
  accC_ref[...] += jnp.sum(onehot, axis=1, keepdims=True)

  @pl.when(i == pl.num_programs(0) - 1)
  def _():
    gden = jnp.maximum(accC_ref[...], 1.0)
    x_class = accD_ref[...] / gden
    x_pool = accJ_ref[...] / gden
    reg_in = jnp.concatenate([other_ref[:, :17], x_class, x_pool], axis=1)
    r = jnp.maximum(jnp.dot(reg_in, Rw1_ref[...],
                            preferred_element_type=jnp.float32)
                    + Rb1_ref[...][None, :], 0.0)
    reg_ref[...] = jnp.dot(r, Rw2_ref[...],
                           preferred_element_type=jnp.float32) \
        + Rb2_ref[...]


def _tc_final(f1, f2, f3, xc, batch3d, other_attrs,
              W1, b1, W2, b2, W3, b3, Rw1, Rb1, Rw2, Rb2):
  grid = N // BT
  return pl.pallas_call(
      _tc_final_body,
      grid=(grid,),
      in_specs=[
          pl.BlockSpec((BT, 128), lambda i: (i, 0)),
          pl.BlockSpec((BT, 128), lambda i: (i, 0)),
          pl.BlockSpec((BT, 128), lambda i: (i, 0)),
          pl.BlockSpec((BT, 1), lambda i: (i, 0)),
          pl.BlockSpec((1, 1, BT), lambda i: (i, 0, 0)),
          pl.BlockSpec((G, 18), lambda i: (0, 0)),
          pl.BlockSpec((384, 256), lambda i: (0, 0)),
          pl.BlockSpec((256,), lambda i: (0,)),
          pl.BlockSpec((256, 64), lambda i: (0, 0)),
          pl.BlockSpec((64,), lambda i: (0,)),
          pl.BlockSpec((64, 1), lambda i: (0, 0)),
          pl.BlockSpec((1, 1), lambda i: (0, 0)),
          pl.BlockSpec((402, 32), lambda i: (0, 0)),
          pl.BlockSpec((32,), lambda i: (0,)),
          pl.BlockSpec((32, 1), lambda i: (0, 0)),
          pl.BlockSpec((1, 1), lambda i: (0, 0)),
      ],
      out_specs=[
          pl.BlockSpec((BT, 1), lambda i: (i, 0)),
          pl.BlockSpec((G, 1), lambda i: (0, 0)),
      ],
      out_shape=[
          jax.ShapeDtypeStruct((N, 1), jnp.float32),
          jax.ShapeDtypeStruct((G, 1), jnp.float32),
      ],
      scratch_shapes=[
          pltpu.VMEM((G, 384), jnp.float32),
          pltpu.VMEM((G, 1), jnp.float32),
          pltpu.VMEM((G, 1), jnp.float32),
      ],
  )(f1, f2, f3, xc, batch3d, other_attrs,
    W1, b1, W2, b2, W3, b3, Rw1, Rb1, Rw2, Rb2)


def kernel(x, edge_index, batch, other_attrs, Wl0, bl0, Wr0, g0, be0,
           Wl1, bl1, Wr1, g1, be1, Wl2, bl2, Wr2, g2, be2,
           W1, b1, W2, b2, W3, b3, Rw1, Rb1, Rw2, Rb2):
  src = edge_index[0].astype(jnp.int32)
  dst = edge_index[1].astype(jnp.int32)
  npad = EPAD - E
  src_p = jnp.concatenate([src, jnp.zeros((npad,), jnp.int32)]
                          ).reshape(EPAD // CHUNK, CHUNK)
  dst_p = jnp.concatenate([dst, jnp.full((npad,), N, jnp.int32)]
                          ).reshape(EPAD // CHUNK, CHUNK)
  srcp2 = jnp.stack([2 * src_p, 2 * src_p + 1])

  def h2d(h):
    return h.reshape(2 * N, 64)

  cnt = _sc_cnt(dst_p)
  aggP0 = _sc_seg(h2d(x), srcp2, dst_p)
  h1 = _tc_layer(aggP0, cnt, x, Wl0, Wr0, bl0, g0, be0)
  aggP1 = _sc_seg(h2d(h1), srcp2, dst_p)
  h2 = _tc_layer(aggP1, cnt, h1, Wl1, Wr1, bl1, g1, be1)
  aggP2 = _sc_seg(h2d(h2), srcp2, dst_p)
  h3 = _tc_layer(aggP2, cnt, h2, Wl2, Wr2, bl2, g2, be2)

  xc = x[:, 0:1]
  batch3d = batch.astype(jnp.int32).reshape(N // BT, 1, BT)
  node_output, reg_output = _tc_final(
      h1, h2, h3, xc, batch3d, other_attrs,
      W1, b1, W2, b2, W3, b3.reshape(1, 1), Rw1, Rb1, Rw2, Rb2.reshape(1, 1))
  last_attr = other_attrs[:, -1:]
  return (node_output, reg_output, last_attr)


# speedup vs baseline: 1.1931x; 1.1931x over previous
import functools
import math

import jax
import jax.numpy as jnp
from jax import lax
from jax.experimental import pallas as pl
from jax.experimental.pallas import tpu as pltpu
from jax.experimental.pallas import tpu_sc as plsc

N = 10000
E = 320000
G = 64

CHUNK = 128
CPT = 160
EPW = CPT * CHUNK
EPAD = 16 * EPW
NROWS = 10240
RPT = NROWS // 16
NB = 5

BN_SCALE = float(1.0 / math.sqrt(1.0 + 1e-5))


def _sc_seg_body(h2_hbm, srcp2_hbm, dstp_hbm, agg_out,
                 src_v, dst_v, rows, agg_sh, gsem, ssem):
  c = lax.axis_index("c")
  s = lax.axis_index("s")

  zbuf = rows[0]

  def _zrow(i, _):
    for k in range(4):
      zbuf[i, pl.ds(k * 16, 16)] = jnp.zeros((16,), jnp.float32)
    return 0

  lax.fori_loop(0, CHUNK, _zrow, 0)
  pltpu.async_copy(srcp2_hbm.at[c, pl.ds(s * CPT, CPT)], src_v, gsem.at[0])
  pltpu.async_copy(dstp_hbm.at[pl.ds(s * CPT, CPT)], dst_v, gsem.at[1])
  for k in range(RPT // CHUNK):
    pltpu.async_copy(zbuf, agg_sh.at[pl.ds(s * RPT + k * CHUNK, CHUNK)],
                     ssem.at[k % NB])
  pltpu.make_async_copy(srcp2_hbm.at[c, pl.ds(s * CPT, CPT)], src_v,
                        gsem.at[0]).wait()
  pltpu.make_async_copy(dstp_hbm.at[pl.ds(s * CPT, CPT)], dst_v,
                        gsem.at[1]).wait()
  for k in range(RPT // CHUNK):
    pltpu.make_async_copy(zbuf, agg_sh.at[pl.ds(s * RPT + k * CHUNK, CHUNK)],
                          ssem.at[k % NB]).wait()
  plsc.subcore_barrier()

  for b in range(NB):
    pltpu.async_copy(h2_hbm.at[src_v.at[b]], rows[b], gsem.at[b])

  def _group(g, _):
    for b in range(NB):
      j = g * NB + b
      pltpu.make_async_copy(h2_hbm.at[src_v.at[j]], rows[b],
                            gsem.at[b]).wait()
      pltpu.async_copy(rows[b], agg_sh.at[dst_v.at[j]], ssem.at[b], add=True)

      @pl.when(j + NB < CPT)
      def _():
        pltpu.make_async_copy(rows[b], agg_sh.at[dst_v.at[j]],
                              ssem.at[b]).wait()
        pltpu.async_copy(h2_hbm.at[src_v.at[j + NB]], rows[b], gsem.at[b])

    return 0

  lax.fori_loop(0, CPT // NB, _group, 0)
  for b in range(NB):
    j = CPT - NB + b
    pltpu.make_async_copy(rows[b], agg_sh.at[dst_v.at[j]], ssem.at[b]).wait()

  plsc.subcore_barrier()

  KOUT = RPT // CHUNK

  def _row(k):
    return s * RPT + k * CHUNK

  for k in range(min(NB, KOUT)):
    pltpu.async_copy(agg_sh.at[pl.ds(_row(k), CHUNK)], rows[k], gsem.at[k])
  for k in range(KOUT):
    b = k % NB
    pltpu.make_async_copy(agg_sh.at[pl.ds(_row(k), CHUNK)], rows[b],
                          gsem.at[b]).wait()
    pltpu.async_copy(rows[b], agg_out.at[c, pl.ds(_row(k), CHUNK)],
                     ssem.at[b])
    if k + NB < KOUT:
      pltpu.make_async_copy(rows[b], agg_out.at[c, pl.ds(_row(k), CHUNK)],
                            ssem.at[b]).wait()
      pltpu.async_copy(agg_sh.at[pl.ds(_row(k + NB), CHUNK)], rows[b],
                       gsem.at[b])
  for k in range(max(KOUT - NB, 0), KOUT):
    b = k % NB
    pltpu.make_async_copy(rows[b], agg_out.at[c, pl.ds(_row(k), CHUNK)],
                          ssem.at[b]).wait()


def _make_sc_seg():
  mesh = plsc.VectorSubcoreMesh(core_axis_name="c", subcore_axis_name="s")
  out_type = jax.ShapeDtypeStruct((2, NROWS, 64), jnp.float32)
  scratch = [
      pltpu.VMEM((CPT, CHUNK), jnp.int32),
      pltpu.VMEM((CPT, CHUNK), jnp.int32),
      [pltpu.VMEM((CHUNK, 64), jnp.float32) for _ in range(NB)],
      pltpu.VMEM_SHARED((NROWS, 64), jnp.float32),
      pltpu.SemaphoreType.DMA((NB,)),
      pltpu.SemaphoreType.DMA((NB,)),
  ]
  return pl.kernel(_sc_seg_body,
                   out_type=out_type, mesh=mesh, scratch_types=scratch,
                   compiler_params=pltpu.CompilerParams(
                       use_tc_tiling_on_sc=False),
                   name="sc_seg")


_sc_seg = _make_sc_seg()

CCPT = 80


def _sc_cnt_body(dstp_hbm, cnt_out, dst_v, ones_v, stage_v, cnt_sh, ssem):
  c = lax.axis_index("c")
  s = lax.axis_index("s")
  wid = c * 16 + s

  def _orow(i, _):
    ones_v[i, :] = jnp.ones((16,), jnp.float32)
    return 0

  lax.fori_loop(0, CHUNK, _orow, 0)

  def _srow(i, _):
    stage_v[i, :] = jnp.zeros((16,), jnp.float32)
    return 0

  lax.fori_loop(0, RPT, _srow, 0)
  pltpu.sync_copy(stage_v, cnt_sh.at[pl.ds(s * RPT, RPT)])
  plsc.subcore_barrier()

  pltpu.sync_copy(dstp_hbm.at[pl.ds(wid * CCPT, CCPT)], dst_v)

  def _grp(g, _):
    for b in range(NB):
      j = g * NB + b
      pltpu.async_copy(ones_v, cnt_sh.at[dst_v.at[j]], ssem.at[b], add=True)
    for b in range(NB):
      j = g * NB + b
      pltpu.make_async_copy(ones_v, cnt_sh.at[dst_v.at[j]],
                            ssem.at[b]).wait()
    return 0

  lax.fori_loop(0, CCPT // NB, _grp, 0)
  plsc.subcore_barrier()

  pltpu.sync_copy(cnt_sh.at[pl.ds(s * RPT, RPT)], stage_v)
  pltpu.sync_copy(stage_v, cnt_out.at[c, pl.ds(s * RPT, RPT)])


def _make_sc_cnt():
  mesh = plsc.VectorSubcoreMesh(core_axis_name="c", subcore_axis_name="s")
  out_type = jax.ShapeDtypeStruct((2, NROWS, 16), jnp.float32)
  scratch = [
      pltpu.VMEM((CCPT, CHUNK), jnp.int32),
      pltpu.VMEM((CHUNK, 16), jnp.float32),
      pltpu.VMEM((RPT, 16), jnp.float32),
      pltpu.VMEM_SHARED((NROWS, 16), jnp.float32),
      pltpu.SemaphoreType.DMA((NB,)),
  ]
  return pl.kernel(_sc_cnt_body,
                   out_type=out_type, mesh=mesh, scratch_types=scratch,
                   compiler_params=pltpu.CompilerParams(
                       use_tc_tiling_on_sc=False),
                   name="sc_cnt")


_sc_cnt = _make_sc_cnt()


BT = 1000


def _tc_layer_body(aggP_ref, cnt_ref, h_ref, Wl_ref, Wr_ref, bl_ref,
                   g_ref, be_ref, out_ref):
  agg = jnp.concatenate([aggP_ref[0], aggP_ref[1]], axis=1)
  cnt = cnt_ref[0, :, 0:1] + cnt_ref[1, :, 0:1]
  denom = jnp.maximum(cnt, 1.0)
  agg = agg / denom
  h = jnp.concatenate([h_ref[0], h_ref[1]], axis=1)
  z = (jnp.dot(agg, Wl_ref[...], preferred_element_type=jnp.float32)
       + bl_ref[...][None, :]
       + jnp.dot(h, Wr_ref[...], preferred_element_type=jnp.float32))
  z = g_ref[...][None, :] * (z * BN_SCALE) + be_ref[...][None, :]
  z = jnp.maximum(z, 0.0)
  out_ref[0] = z[:, :64]
  out_ref[1] = z[:, 64:]


def _tc_layer(aggP, cnt, hsplit, Wl, Wr, bl, g, be):
  grid = N // BT
  return pl.pallas_call(
      _tc_layer_body,
      grid=(grid,),
      in_specs=[
          pl.BlockSpec((2, BT, 64), lambda i: (0, i, 0)),
          pl.BlockSpec((2, BT, 16), lambda i: (0, i, 0)),
          pl.BlockSpec((2, BT, 64), lambda i: (0, i, 0)),
          pl.BlockSpec((128, 128), lambda i: (0, 0)),
          pl.BlockSpec((128, 128), lambda i: (0, 0)),
          pl.BlockSpec((128,), lambda i: (0,)),
          pl.BlockSpec((128,), lambda i: (0,)),
          pl.BlockSpec((128,), lambda i: (0,)),
      ],
      out_specs=pl.BlockSpec((2, BT, 64), lambda i: (0, i, 0)),
      out_shape=jax.ShapeDtypeStruct((2, N, 64), jnp.float32),
  )(aggP, cnt, hsplit, Wl, Wr, bl, g, be)


def _tc_final_body(f1_ref, f2_ref, f3_ref, xc_ref, batch_ref, other_ref,
                   W1_ref, b1_ref, W2_ref, b2_ref, W3_ref, b3_ref,
                   Rw1_ref, Rb1_ref, Rw2_ref, Rb2_ref,
                   no_ref, reg_ref, accJ_ref, accD_ref, accC_ref):
  i = pl.program_id(0)
  jk = jnp.concatenate([f1_ref[0], f1_ref[1], f2_ref[0], f2_ref[1],
                        f3_ref[0], f3_ref[1]], axis=1)
  t = jnp.maximum(jnp.dot(jk, W1_ref[...],
                          preferred_element_type=jnp.float32)
                  + b1_ref[...][None, :], 0.0)
  t = jnp.maximum(jnp.dot(t, W2_ref[...],
                          preferred_element_type=jnp.float32)
                  + b2_ref[...][None, :], 0.0)
  no = jnp.dot(t, W3_ref[...], preferred_element_type=jnp.float32) \
      + b3_ref[...]
  no_ref[...] = no
  delay = no * xc_ref[...]

  onehot = (lax.broadcasted_iota(jnp.int32, (G, BT), 0)
            == batch_ref[0, 0][None, :]).astype(jnp.float32)

  @pl.when(i == 0)
  def _():
    accJ_ref[...] = jnp.zeros_like(accJ_ref)
    accD_ref[...] = jnp.zeros_like(accD_ref)
    accC_ref[...] = jnp.zeros_like(accC_ref)

  accJ_ref[...] += jnp.dot(onehot, jk, preferred_element_type=jnp.float32)
  accD_ref[...] += jnp.dot(onehot, delay, preferred_element_type=jnp.float32)
  accC_ref[...] += jnp.sum(onehot, axis=1, keepdims=True)

  @pl.when(i == pl.num_programs(0) - 1)
  def _():
    gden = jnp.maximum(accC_ref[...], 1.0)
    x_class = accD_ref[...] / gden
    x_pool = accJ_ref[...] / gden
    reg_in = jnp.concatenate([other_ref[:, :17], x_class, x_pool], axis=1)
    r = jnp.maximum(jnp.dot(reg_in, Rw1_ref[...],
                            preferred_element_type=jnp.float32)
                    + Rb1_ref[...][None, :], 0.0)
    reg_ref[...] = jnp.dot(r, Rw2_ref[...],
                           preferred_element_type=jnp.float32) \
        + Rb2_ref[...]


def _tc_final(f1, f2, f3, xc, batch3d, other_attrs,
              W1, b1, W2, b2, W3, b3, Rw1, Rb1, Rw2, Rb2):
  grid = N // BT
  return pl.pallas_call(
      _tc_final_body,
      grid=(grid,),
      in_specs=[
          pl.BlockSpec((2, BT, 64), lambda i: (0, i, 0)),
          pl.BlockSpec((2, BT, 64), lambda i: (0, i, 0)),
          pl.BlockSpec((2, BT, 64), lambda i: (0, i, 0)),
          pl.BlockSpec((BT, 1), lambda i: (i, 0)),
          pl.BlockSpec((1, 1, BT), lambda i: (i, 0, 0)),
          pl.BlockSpec((G, 18), lambda i: (0, 0)),
          pl.BlockSpec((384, 256), lambda i: (0, 0)),
          pl.BlockSpec((256,), lambda i: (0,)),
          pl.BlockSpec((256, 64), lambda i: (0, 0)),
          pl.BlockSpec((64,), lambda i: (0,)),
          pl.BlockSpec((64, 1), lambda i: (0, 0)),
          pl.BlockSpec((1, 1), lambda i: (0, 0)),
          pl.BlockSpec((402, 32), lambda i: (0, 0)),
          pl.BlockSpec((32,), lambda i: (0,)),
          pl.BlockSpec((32, 1), lambda i: (0, 0)),
          pl.BlockSpec((1, 1), lambda i: (0, 0)),
      ],
      out_specs=[
          pl.BlockSpec((BT, 1), lambda i: (i, 0)),
          pl.BlockSpec((G, 1), lambda i: (0, 0)),
      ],
      out_shape=[
          jax.ShapeDtypeStruct((N, 1), jnp.float32),
          jax.ShapeDtypeStruct((G, 1), jnp.float32),
      ],
      scratch_shapes=[
          pltpu.VMEM((G, 384), jnp.float32),
          pltpu.VMEM((G, 1), jnp.float32),
          pltpu.VMEM((G, 1), jnp.float32),
      ],
  )(f1, f2, f3, xc, batch3d, other_attrs,
    W1, b1, W2, b2, W3, b3, Rw1, Rb1, Rw2, Rb2)


def kernel(x, edge_index, batch, other_attrs, Wl0, bl0, Wr0, g0, be0,
           Wl1, bl1, Wr1, g1, be1, Wl2, bl2, Wr2, g2, be2,
           W1, b1, W2, b2, W3, b3, Rw1, Rb1, Rw2, Rb2):
  src = edge_index[0].astype(jnp.int32)
  dst = edge_index[1].astype(jnp.int32)
  npad = EPAD - E
  src_p = jnp.concatenate([src, jnp.zeros((npad,), jnp.int32)]
                          ).reshape(EPAD // CHUNK, CHUNK)
  dst_p = jnp.concatenate([dst, jnp.full((npad,), N, jnp.int32)]
                          ).reshape(EPAD // CHUNK, CHUNK)
  srcp2 = jnp.stack([src_p, src_p + N])

  xsplit = jnp.stack([x[:, :64], x[:, 64:]])

  def h2d(hs):
    return hs.reshape(2 * N, 64)

  cnt = _sc_cnt(dst_p)
  aggP0 = _sc_seg(h2d(xsplit), srcp2, dst_p)
  h1 = _tc_layer(aggP0, cnt, xsplit, Wl0, Wr0, bl0, g0, be0)
  aggP1 = _sc_seg(h2d(h1), srcp2, dst_p)
  h2 = _tc_layer(aggP1, cnt, h1, Wl1, Wr1, bl1, g1, be1)
  aggP2 = _sc_seg(h2d(h2), srcp2, dst_p)
  h3 = _tc_layer(aggP2, cnt, h2, Wl2, Wr2, bl2, g2, be2)

  xc = x[:, 0:1]
  batch3d = batch.astype(jnp.int32).reshape(N // BT, 1, BT)
  node_output, reg_output = _tc_final(
      h1, h2, h3, xc, batch3d, other_attrs,
      W1, b1, W2, b2, W3, b3.reshape(1, 1), Rw1, Rb1, Rw2, Rb2.reshape(1, 1))
  last_attr = other_attrs[:, -1:]
  return (node_output, reg_output, last_attr)


# speedup vs baseline: 2.3233x; 1.9472x over previous
import functools
import math

import jax
import jax.numpy as jnp
from jax import lax
from jax.experimental import pallas as pl
from jax.experimental.pallas import tpu as pltpu
from jax.experimental.pallas import tpu_sc as plsc

N = 10000
E = 320000
G = 64

CHUNK = 128
CPT = 157
EPW = CPT * CHUNK
EPAD = 16 * EPW
NROWS = 10240
RPT = NROWS // 16
NB = 5

BN_SCALE = float(1.0 / math.sqrt(1.0 + 1e-5))


def _sc_seg_body(h2_hbm, srcp2_hbm, dstp_hbm, agg_out,
                 src_v, dst_v, rows, agg_sh, gsem, ssem):
  c = lax.axis_index("c")
  s = lax.axis_index("s")

  zbuf = rows[0]

  def _zrow(i, _):
    for k in range(4):
      zbuf[i, pl.ds(k * 16, 16)] = jnp.zeros((16,), jnp.float32)
    return 0

  lax.fori_loop(0, CHUNK, _zrow, 0)
  pltpu.async_copy(srcp2_hbm.at[c, pl.ds(s * CPT, CPT)], src_v, gsem.at[0])
  pltpu.async_copy(dstp_hbm.at[pl.ds(s * CPT, CPT)], dst_v, gsem.at[1])
  for k in range(RPT // CHUNK):
    pltpu.async_copy(zbuf, agg_sh.at[pl.ds(s * RPT + k * CHUNK, CHUNK)],
                     ssem.at[k % NB])
  pltpu.make_async_copy(srcp2_hbm.at[c, pl.ds(s * CPT, CPT)], src_v,
                        gsem.at[0]).wait()
  pltpu.make_async_copy(dstp_hbm.at[pl.ds(s * CPT, CPT)], dst_v,
                        gsem.at[1]).wait()
  for k in range(RPT // CHUNK):
    pltpu.make_async_copy(zbuf, agg_sh.at[pl.ds(s * RPT + k * CHUNK, CHUNK)],
                          ssem.at[k % NB]).wait()
  for b in range(NB):
    pltpu.async_copy(h2_hbm.at[src_v.at[b]], rows[b], gsem.at[b])
  plsc.subcore_barrier()

  def _group(g, _):
    for b in range(NB):
      j = g * NB + b

      @pl.when(j < CPT)
      def _():
        pltpu.make_async_copy(h2_hbm.at[src_v.at[j]], rows[b],
                              gsem.at[b]).wait()
        pltpu.async_copy(rows[b], agg_sh.at[dst_v.at[j]], ssem.at[b],
                         add=True)

        @pl.when(j + NB < CPT)
        def _():
          pltpu.make_async_copy(rows[b], agg_sh.at[dst_v.at[j]],
                                ssem.at[b]).wait()
          pltpu.async_copy(h2_hbm.at[src_v.at[j + NB]], rows[b], gsem.at[b])

    return 0

  lax.fori_loop(0, (CPT + NB - 1) // NB, _group, 0)
  for j in range(CPT - NB, CPT):
    pltpu.make_async_copy(rows[j % NB], agg_sh.at[dst_v.at[j]],
                          ssem.at[j % NB]).wait()

  plsc.subcore_barrier()

  KOUT = RPT // CHUNK

  def _row(k):
    return s * RPT + k * CHUNK

  for k in range(min(NB, KOUT)):
    pltpu.async_copy(agg_sh.at[pl.ds(_row(k), CHUNK)], rows[k], gsem.at[k])
  for k in range(KOUT):
    b = k % NB
    pltpu.make_async_copy(agg_sh.at[pl.ds(_row(k), CHUNK)], rows[b],
                          gsem.at[b]).wait()
    pltpu.async_copy(rows[b], agg_out.at[c, pl.ds(_row(k), CHUNK)],
                     ssem.at[b])
    if k + NB < KOUT:
      pltpu.make_async_copy(rows[b], agg_out.at[c, pl.ds(_row(k), CHUNK)],
                            ssem.at[b]).wait()
      pltpu.async_copy(agg_sh.at[pl.ds(_row(k + NB), CHUNK)], rows[b],
                       gsem.at[b])
  for k in range(max(KOUT - NB, 0), KOUT):
    b = k % NB
    pltpu.make_async_copy(rows[b], agg_out.at[c, pl.ds(_row(k), CHUNK)],
                          ssem.at[b]).wait()


def _make_sc_seg():
  mesh = plsc.VectorSubcoreMesh(core_axis_name="c", subcore_axis_name="s")
  out_type = jax.ShapeDtypeStruct((2, NROWS, 64), jnp.float32)
  scratch = [
      pltpu.VMEM((CPT, CHUNK), jnp.int32),
      pltpu.VMEM((CPT, CHUNK), jnp.int32),
      [pltpu.VMEM((CHUNK, 64), jnp.float32) for _ in range(NB)],
      pltpu.VMEM_SHARED((NROWS, 64), jnp.float32),
      pltpu.SemaphoreType.DMA((NB,)),
      pltpu.SemaphoreType.DMA((NB,)),
  ]
  return pl.kernel(_sc_seg_body,
                   out_type=out_type, mesh=mesh, scratch_types=scratch,
                   compiler_params=pltpu.CompilerParams(
                       use_tc_tiling_on_sc=False),
                   name="sc_seg")


_sc_seg = _make_sc_seg()

CHALF = (CPT + 1) // 2


def _sc_cnt_body(dstp_hbm, cnt_out, dst_v, ones_v, stage_v, cnt_sh, ssem):
  c = lax.axis_index("c")
  s = lax.axis_index("s")

  def _orow(i, _):
    ones_v[i, :] = jnp.ones((16,), jnp.float32)
    return 0

  lax.fori_loop(0, CHUNK, _orow, 0)

  def _srow(i, _):
    stage_v[i, :] = jnp.zeros((16,), jnp.float32)
    return 0

  lax.fori_loop(0, RPT, _srow, 0)
  pltpu.sync_copy(stage_v, cnt_sh.at[pl.ds(s * RPT, RPT)])
  plsc.subcore_barrier()

  pltpu.sync_copy(dstp_hbm.at[pl.ds(s * CPT, CPT)], dst_v)

  def _mine(j):
    return (j < CHALF) == (c == 0)

  def _grp(g, _):
    for b in range(NB):
      j = g * NB + b

      @pl.when(jnp.logical_and(j < CPT, _mine(j)))
      def _():
        pltpu.async_copy(ones_v, cnt_sh.at[dst_v.at[j]], ssem.at[b],
                         add=True)
    for b in range(NB):
      j = g * NB + b

      @pl.when(jnp.logical_and(j < CPT, _mine(j)))
      def _():
        pltpu.make_async_copy(ones_v, cnt_sh.at[dst_v.at[j]],
                              ssem.at[b]).wait()
    return 0

  lax.fori_loop(0, (CPT + NB - 1) // NB, _grp, 0)
  plsc.subcore_barrier()

  pltpu.sync_copy(cnt_sh.at[pl.ds(s * RPT, RPT)], stage_v)
  pltpu.sync_copy(stage_v, cnt_out.at[c, pl.ds(s * RPT, RPT)])


def _make_sc_cnt():
  mesh = plsc.VectorSubcoreMesh(core_axis_name="c", subcore_axis_name="s")
  out_type = jax.ShapeDtypeStruct((2, NROWS, 16), jnp.float32)
  scratch = [
      pltpu.VMEM((CPT, CHUNK), jnp.int32),
      pltpu.VMEM((CHUNK, 16), jnp.float32),
      pltpu.VMEM((RPT, 16), jnp.float32),
      pltpu.VMEM_SHARED((NROWS, 16), jnp.float32),
      pltpu.SemaphoreType.DMA((NB,)),
  ]
  return pl.kernel(_sc_cnt_body,
                   out_type=out_type, mesh=mesh, scratch_types=scratch,
                   compiler_params=pltpu.CompilerParams(
                       use_tc_tiling_on_sc=False),
                   name="sc_cnt")


_sc_cnt = _make_sc_cnt()


BT = 1000


def _tc_layer_body(aggP_ref, cnt_ref, h_ref, Wl_ref, Wr_ref, bl_ref,
                   g_ref, be_ref, out_ref):
  agg = jnp.concatenate([aggP_ref[0], aggP_ref[1]], axis=1)
  cnt = cnt_ref[0, :, 0:1] + cnt_ref[1, :, 0:1]
  denom = jnp.maximum(cnt, 1.0)
  agg = agg / denom
  h = jnp.concatenate([h_ref[0], h_ref[1]], axis=1)
  z = (jnp.dot(agg, Wl_ref[...], preferred_element_type=jnp.float32)
       + bl_ref[...][None, :]
       + jnp.dot(h, Wr_ref[...], preferred_element_type=jnp.float32))
  z = g_ref[...][None, :] * (z * BN_SCALE) + be_ref[...][None, :]
  z = jnp.maximum(z, 0.0)
  out_ref[0] = z[:, :64]
  out_ref[1] = z[:, 64:]


def _tc_layer(aggP, cnt, hsplit, Wl, Wr, bl, g, be):
  grid = N // BT
  return pl.pallas_call(
      _tc_layer_body,
      grid=(grid,),
      in_specs=[
          pl.BlockSpec((2, BT, 64), lambda i: (0, i, 0)),
          pl.BlockSpec((2, BT, 16), lambda i: (0, i, 0)),
          pl.BlockSpec((2, BT, 64), lambda i: (0, i, 0)),
          pl.BlockSpec((128, 128), lambda i: (0, 0)),
          pl.BlockSpec((128, 128), lambda i: (0, 0)),
          pl.BlockSpec((128,), lambda i: (0,)),
          pl.BlockSpec((128,), lambda i: (0,)),
          pl.BlockSpec((128,), lambda i: (0,)),
      ],
      out_specs=pl.BlockSpec((2, BT, 64), lambda i: (0, i, 0)),
      out_shape=jax.ShapeDtypeStruct((2, N, 64), jnp.float32),
  )(aggP, cnt, hsplit, Wl, Wr, bl, g, be)


def _tc_final_body(aggP_ref, cnt_ref, h2s_ref, Wl_ref, Wr_ref, bl_ref,
                   g_ref, be_ref, f1_ref, xc_ref, batch_ref, other_ref,
                   W1_ref, b1_ref, W2_ref, b2_ref, W3_ref, b3_ref,
                   Rw1_ref, Rb1_ref, Rw2_ref, Rb2_ref,
                   no_ref, reg_ref, accJ_ref, accD_ref, accC_ref):
  i = pl.program_id(0)
  agg = jnp.concatenate([aggP_ref[0], aggP_ref[1]], axis=1)
  cnt = cnt_ref[0, :, 0:1] + cnt_ref[1, :, 0:1]
  agg = agg / jnp.maximum(cnt, 1.0)
  h2 = jnp.concatenate([h2s_ref[0], h2s_ref[1]], axis=1)
  z = (jnp.dot(agg, Wl_ref[...], preferred_element_type=jnp.float32)
       + bl_ref[...][None, :]
       + jnp.dot(h2, Wr_ref[...], preferred_element_type=jnp.float32))
  z = g_ref[...][None, :] * (z * BN_SCALE) + be_ref[...][None, :]
  h3 = jnp.maximum(z, 0.0)

  jk = jnp.concatenate([f1_ref[0], f1_ref[1], h2s_ref[0], h2s_ref[1], h3],
                       axis=1)
  t = jnp.maximum(jnp.dot(jk, W1_ref[...],
                          preferred_element_type=jnp.float32)
                  + b1_ref[...][None, :], 0.0)
  t = jnp.maximum(jnp.dot(t, W2_ref[...],
                          preferred_element_type=jnp.float32)
                  + b2_ref[...][None, :], 0.0)
  no = jnp.dot(t, W3_ref[...], preferred_element_type=jnp.float32) \
      + b3_ref[...]
  no_ref[...] = no
  delay = no * xc_ref[...]

  onehot = (lax.broadcasted_iota(jnp.int32, (G, BT), 0)
            == batch_ref[0, 0][None, :]).astype(jnp.float32)

  @pl.when(i == 0)
  def _():
    accJ_ref[...] = jnp.zeros_like(accJ_ref)
    accD_ref[...] = jnp.zeros_like(accD_ref)
    accC_ref[...] = jnp.zeros_like(accC_ref)

  accJ_ref[...] += jnp.dot(onehot, jk, preferred_element_type=jnp.float32)
  accD_ref[...] += jnp.dot(onehot, delay, preferred_element_type=jnp.float32)
  accC_ref[...] += jnp.sum(onehot, axis=1, keepdims=True)

  @pl.when(i == pl.num_programs(0) - 1)
  def _():
    gden = jnp.maximum(accC_ref[...], 1.0)
    x_class = accD_ref[...] / gden
    x_pool = accJ_ref[...] / gden
    reg_in = jnp.concatenate([other_ref[:, :17], x_class, x_pool], axis=1)
    r = jnp.maximum(jnp.dot(reg_in, Rw1_ref[...],
                            preferred_element_type=jnp.float32)
                    + Rb1_ref[...][None, :], 0.0)
    reg_ref[...] = jnp.dot(r, Rw2_ref[...],
                           preferred_element_type=jnp.float32) \
        + Rb2_ref[...]


def _tc_final(aggP, cnt, h2s, Wl, Wr, bl, g, be, f1, xc, batch3d,
              other_attrs, W1, b1, W2, b2, W3, b3, Rw1, Rb1, Rw2, Rb2):
  grid = N // BT
  return pl.pallas_call(
      _tc_final_body,
      grid=(grid,),
      in_specs=[
          pl.BlockSpec((2, BT, 64), lambda i: (0, i, 0)),
          pl.BlockSpec((2, BT, 16), lambda i: (0, i, 0)),
          pl.BlockSpec((2, BT, 64), lambda i: (0, i, 0)),
          pl.BlockSpec((128, 128), lambda i: (0, 0)),
          pl.BlockSpec((128, 128), lambda i: (0, 0)),
          pl.BlockSpec((128,), lambda i: (0,)),
          pl.BlockSpec((128,), lambda i: (0,)),
          pl.BlockSpec((128,), lambda i: (0,)),
          pl.BlockSpec((2, BT, 64), lambda i: (0, i, 0)),
          pl.BlockSpec((BT, 1), lambda i: (i, 0)),
          pl.BlockSpec((1, 1, BT), lambda i: (i, 0, 0)),
          pl.BlockSpec((G, 18), lambda i: (0, 0)),
          pl.BlockSpec((384, 256), lambda i: (0, 0)),
          pl.BlockSpec((256,), lambda i: (0,)),
          pl.BlockSpec((256, 64), lambda i: (0, 0)),
          pl.BlockSpec((64,), lambda i: (0,)),
          pl.BlockSpec((64, 1), lambda i: (0, 0)),
          pl.BlockSpec((1, 1), lambda i: (0, 0)),
          pl.BlockSpec((402, 32), lambda i: (0, 0)),
          pl.BlockSpec((32,), lambda i: (0,)),
          pl.BlockSpec((32, 1), lambda i: (0, 0)),
          pl.BlockSpec((1, 1), lambda i: (0, 0)),
      ],
      out_specs=[
          pl.BlockSpec((BT, 1), lambda i: (i, 0)),
          pl.BlockSpec((G, 1), lambda i: (0, 0)),
      ],
      out_shape=[
          jax.ShapeDtypeStruct((N, 1), jnp.float32),
          jax.ShapeDtypeStruct((G, 1), jnp.float32),
      ],
      scratch_shapes=[
          pltpu.VMEM((G, 384), jnp.float32),
          pltpu.VMEM((G, 1), jnp.float32),
          pltpu.VMEM((G, 1), jnp.float32),
      ],
  )(aggP, cnt, h2s, Wl, Wr, bl, g, be, f1, xc, batch3d, other_attrs,
    W1, b1, W2, b2, W3, b3, Rw1, Rb1, Rw2, Rb2)


def kernel(x, edge_index, batch, other_attrs, Wl0, bl0, Wr0, g0, be0,
           Wl1, bl1, Wr1, g1, be1, Wl2, bl2, Wr2, g2, be2,
           W1, b1, W2, b2, W3, b3, Rw1, Rb1, Rw2, Rb2):
  src = edge_index[0].astype(jnp.int32)
  dst = edge_index[1].astype(jnp.int32)
  npad = EPAD - E
  src_p = jnp.concatenate([src, jnp.zeros((npad,), jnp.int32)]
                          ).reshape(EPAD // CHUNK, CHUNK)
  dst_p = jnp.concatenate([dst, jnp.full((npad,), N, jnp.int32)]
                          ).reshape(EPAD // CHUNK, CHUNK)
  srcp2 = jnp.stack([src_p, src_p + N])

  xsplit = jnp.stack([x[:, :64], x[:, 64:]])

  def h2d(hs):
    return hs.reshape(2 * N, 64)

  cnt = _sc_cnt(dst_p)
  aggP0 = _sc_seg(h2d(xsplit), srcp2, dst_p)
  h1 = _tc_layer(aggP0, cnt, xsplit, Wl0, Wr0, bl0, g0, be0)
  aggP1 = _sc_seg(h2d(h1), srcp2, dst_p)
  h2 = _tc_layer(aggP1, cnt, h1, Wl1, Wr1, bl1, g1, be1)
  aggP2 = _sc_seg(h2d(h2), srcp2, dst_p)

  xc = x[:, 0:1]
  batch3d = batch.astype(jnp.int32).reshape(N // BT, 1, BT)
  node_output, reg_output = _tc_final(
      aggP2, cnt, h2, Wl2, Wr2, bl2, g2, be2, h1, xc, batch3d, other_attrs,
      W1, b1, W2, b2, W3, b3.reshape(1, 1), Rw1, Rb1, Rw2, Rb2.reshape(1, 1))
  last_attr = other_attrs[:, -1:]
  return (node_output, reg_output, last_attr)


# speedup vs baseline: 2.3546x; 1.0135x over previous
import functools
import math

import jax
import jax.numpy as jnp
from jax import lax
from jax.experimental import pallas as pl
from jax.experimental.pallas import tpu as pltpu
from jax.experimental.pallas import tpu_sc as plsc

N = 10000
E = 320000
G = 64

CHUNK = 128
CPT = 157
EPW = CPT * CHUNK
EPAD = 16 * EPW
NROWS = 10240
RPT = NROWS // 16
NB = 6

BN_SCALE = float(1.0 / math.sqrt(1.0 + 1e-5))


def _sc_seg_body(h2_hbm, srcp2_hbm, dstp_hbm, agg_out,
                 src_v, dst_v, rows, agg_sh, gsem, ssem):
  c = lax.axis_index("c")
  s = lax.axis_index("s")

  zbuf = rows[0]

  def _zrow(i, _):
    for k in range(4):
      zbuf[i, pl.ds(k * 16, 16)] = jnp.zeros((16,), jnp.float32)
    return 0

  lax.fori_loop(0, CHUNK, _zrow, 0)
  pltpu.async_copy(srcp2_hbm.at[c, pl.ds(s * CPT, CPT)], src_v, gsem.at[0])
  pltpu.async_copy(dstp_hbm.at[pl.ds(s * CPT, CPT)], dst_v, gsem.at[1])
  for k in range(RPT // CHUNK):
    pltpu.async_copy(zbuf, agg_sh.at[pl.ds(s * RPT + k * CHUNK, CHUNK)],
                     ssem.at[k % NB])
  pltpu.make_async_copy(srcp2_hbm.at[c, pl.ds(s * CPT, CPT)], src_v,
                        gsem.at[0]).wait()
  pltpu.make_async_copy(dstp_hbm.at[pl.ds(s * CPT, CPT)], dst_v,
                        gsem.at[1]).wait()
  for k in range(RPT // CHUNK):
    pltpu.make_async_copy(zbuf, agg_sh.at[pl.ds(s * RPT + k * CHUNK, CHUNK)],
                          ssem.at[k % NB]).wait()
  for b in range(NB):
    pltpu.async_copy(h2_hbm.at[src_v.at[b]], rows[b], gsem.at[b])
  plsc.subcore_barrier()

  def _group(g, _):
    for b in range(NB):
      j = g * NB + b

      @pl.when(j < CPT)
      def _():
        pltpu.make_async_copy(h2_hbm.at[src_v.at[j]], rows[b],
                              gsem.at[b]).wait()
        pltpu.async_copy(rows[b], agg_sh.at[dst_v.at[j]], ssem.at[b],
                         add=True)

        @pl.when(j + NB < CPT)
        def _():
          pltpu.make_async_copy(rows[b], agg_sh.at[dst_v.at[j]],
                                ssem.at[b]).wait()
          pltpu.async_copy(h2_hbm.at[src_v.at[j + NB]], rows[b], gsem.at[b])

    return 0

  lax.fori_loop(0, (CPT + NB - 1) // NB, _group, 0)
  for j in range(CPT - NB, CPT):
    pltpu.make_async_copy(rows[j % NB], agg_sh.at[dst_v.at[j]],
                          ssem.at[j % NB]).wait()

  plsc.subcore_barrier()

  KOUT = RPT // CHUNK

  def _row(k):
    return s * RPT + k * CHUNK

  for k in range(min(NB, KOUT)):
    pltpu.async_copy(agg_sh.at[pl.ds(_row(k), CHUNK)], rows[k], gsem.at[k])
  for k in range(KOUT):
    b = k % NB
    pltpu.make_async_copy(agg_sh.at[pl.ds(_row(k), CHUNK)], rows[b],
                          gsem.at[b]).wait()
    pltpu.async_copy(rows[b], agg_out.at[c, pl.ds(_row(k), CHUNK)],
                     ssem.at[b])
    if k + NB < KOUT:
      pltpu.make_async_copy(rows[b], agg_out.at[c, pl.ds(_row(k), CHUNK)],
                            ssem.at[b]).wait()
      pltpu.async_copy(agg_sh.at[pl.ds(_row(k + NB), CHUNK)], rows[b],
                       gsem.at[b])
  for k in range(max(KOUT - NB, 0), KOUT):
    b = k % NB
    pltpu.make_async_copy(rows[b], agg_out.at[c, pl.ds(_row(k), CHUNK)],
                          ssem.at[b]).wait()


def _make_sc_seg():
  mesh = plsc.VectorSubcoreMesh(core_axis_name="c", subcore_axis_name="s")
  out_type = jax.ShapeDtypeStruct((2, NROWS, 64), jnp.float32)
  scratch = [
      pltpu.VMEM((CPT, CHUNK), jnp.int32),
      pltpu.VMEM((CPT, CHUNK), jnp.int32),
      [pltpu.VMEM((CHUNK, 64), jnp.float32) for _ in range(NB)],
      pltpu.VMEM_SHARED((NROWS, 64), jnp.float32),
      pltpu.SemaphoreType.DMA((NB,)),
      pltpu.SemaphoreType.DMA((NB,)),
  ]
  return pl.kernel(_sc_seg_body,
                   out_type=out_type, mesh=mesh, scratch_types=scratch,
                   compiler_params=pltpu.CompilerParams(
                       use_tc_tiling_on_sc=False),
                   name="sc_seg")


_sc_seg = _make_sc_seg()

CHALF = (CPT + 1) // 2


def _sc_cnt_body(dstp_hbm, cnt_out, dst_v, ones_v, stage_v, cnt_sh, ssem):
  c = lax.axis_index("c")
  s = lax.axis_index("s")

  def _orow(i, _):
    ones_v[i, :] = jnp.ones((16,), jnp.float32)
    return 0

  lax.fori_loop(0, CHUNK, _orow, 0)

  def _srow(i, _):
    stage_v[i, :] = jnp.zeros((16,), jnp.float32)
    return 0

  lax.fori_loop(0, RPT, _srow, 0)
  pltpu.sync_copy(stage_v, cnt_sh.at[pl.ds(s * RPT, RPT)])
  plsc.subcore_barrier()

  pltpu.sync_copy(dstp_hbm.at[pl.ds(s * CPT, CPT)], dst_v)

  def _mine(j):
    return (j < CHALF) == (c == 0)

  def _grp(g, _):
    for b in range(NB):
      j = g * NB + b

      @pl.when(jnp.logical_and(j < CPT, _mine(j)))
      def _():
        pltpu.async_copy(ones_v, cnt_sh.at[dst_v.at[j]], ssem.at[b],
                         add=True)
    for b in range(NB):
      j = g * NB + b

      @pl.when(jnp.logical_and(j < CPT, _mine(j)))
      def _():
        pltpu.make_async_copy(ones_v, cnt_sh.at[dst_v.at[j]],
                              ssem.at[b]).wait()
    return 0

  lax.fori_loop(0, (CPT + NB - 1) // NB, _grp, 0)
  plsc.subcore_barrier()

  pltpu.sync_copy(cnt_sh.at[pl.ds(s * RPT, RPT)], stage_v)
  pltpu.sync_copy(stage_v, cnt_out.at[c, pl.ds(s * RPT, RPT)])


def _make_sc_cnt():
  mesh = plsc.VectorSubcoreMesh(core_axis_name="c", subcore_axis_name="s")
  out_type = jax.ShapeDtypeStruct((2, NROWS, 16), jnp.float32)
  scratch = [
      pltpu.VMEM((CPT, CHUNK), jnp.int32),
      pltpu.VMEM((CHUNK, 16), jnp.float32),
      pltpu.VMEM((RPT, 16), jnp.float32),
      pltpu.VMEM_SHARED((NROWS, 16), jnp.float32),
      pltpu.SemaphoreType.DMA((NB,)),
  ]
  return pl.kernel(_sc_cnt_body,
                   out_type=out_type, mesh=mesh, scratch_types=scratch,
                   compiler_params=pltpu.CompilerParams(
                       use_tc_tiling_on_sc=False),
                   name="sc_cnt")


_sc_cnt = _make_sc_cnt()


BT = 2000


def _tc_layer_body(aggP_ref, cnt_ref, h_ref, Wl_ref, Wr_ref, bl_ref,
                   g_ref, be_ref, out_ref):
  agg = jnp.concatenate([aggP_ref[0], aggP_ref[1]], axis=1)
  cnt = cnt_ref[0, :, 0:1] + cnt_ref[1, :, 0:1]
  denom = jnp.maximum(cnt, 1.0)
  agg = agg / denom
  h = jnp.concatenate([h_ref[0], h_ref[1]], axis=1)
  z = (jnp.dot(agg, Wl_ref[...], preferred_element_type=jnp.float32)
       + bl_ref[...][None, :]
       + jnp.dot(h, Wr_ref[...], preferred_element_type=jnp.float32))
  z = g_ref[...][None, :] * (z * BN_SCALE) + be_ref[...][None, :]
  z = jnp.maximum(z, 0.0)
  out_ref[0] = z[:, :64]
  out_ref[1] = z[:, 64:]


def _tc_layer(aggP, cnt, hsplit, Wl, Wr, bl, g, be):
  grid = N // BT
  return pl.pallas_call(
      _tc_layer_body,
      grid=(grid,),
      in_specs=[
          pl.BlockSpec((2, BT, 64), lambda i: (0, i, 0)),
          pl.BlockSpec((2, BT, 16), lambda i: (0, i, 0)),
          pl.BlockSpec((2, BT, 64), lambda i: (0, i, 0)),
          pl.BlockSpec((128, 128), lambda i: (0, 0)),
          pl.BlockSpec((128, 128), lambda i: (0, 0)),
          pl.BlockSpec((128,), lambda i: (0,)),
          pl.BlockSpec((128,), lambda i: (0,)),
          pl.BlockSpec((128,), lambda i: (0,)),
      ],
      out_specs=pl.BlockSpec((2, BT, 64), lambda i: (0, i, 0)),
      out_shape=jax.ShapeDtypeStruct((2, N, 64), jnp.float32),
  )(aggP, cnt, hsplit, Wl, Wr, bl, g, be)


def _tc_final_body(aggP_ref, cnt_ref, h2s_ref, Wl_ref, Wr_ref, bl_ref,
                   g_ref, be_ref, f1_ref, xc_ref, batch_ref, other_ref,
                   W1_ref, b1_ref, W2_ref, b2_ref, W3_ref, b3_ref,
                   Rw1_ref, Rb1_ref, Rw2_ref, Rb2_ref,
                   no_ref, reg_ref, accJ_ref, accD_ref, accC_ref):
  i = pl.program_id(0)
  agg = jnp.concatenate([aggP_ref[0], aggP_ref[1]], axis=1)
  cnt = cnt_ref[0, :, 0:1] + cnt_ref[1, :, 0:1]
  agg = agg / jnp.maximum(cnt, 1.0)
  h2 = jnp.concatenate([h2s_ref[0], h2s_ref[1]], axis=1)
  z = (jnp.dot(agg, Wl_ref[...], preferred_element_type=jnp.float32)
       + bl_ref[...][None, :]
       + jnp.dot(h2, Wr_ref[...], preferred_element_type=jnp.float32))
  z = g_ref[...][None, :] * (z * BN_SCALE) + be_ref[...][None, :]
  h3 = jnp.maximum(z, 0.0)

  jk = jnp.concatenate([f1_ref[0], f1_ref[1], h2s_ref[0], h2s_ref[1], h3],
                       axis=1)
  t = jnp.maximum(jnp.dot(jk, W1_ref[...],
                          preferred_element_type=jnp.float32)
                  + b1_ref[...][None, :], 0.0)
  t = jnp.maximum(jnp.dot(t, W2_ref[...],
                          preferred_element_type=jnp.float32)
                  + b2_ref[...][None, :], 0.0)
  no = jnp.dot(t, W3_ref[...], preferred_element_type=jnp.float32) \
      + b3_ref[...]
  no_ref[...] = no
  delay = no * xc_ref[...]

  onehot = (lax.broadcasted_iota(jnp.int32, (G, BT), 0)
            == batch_ref[0, 0][None, :]).astype(jnp.float32)

  @pl.when(i == 0)
  def _():
    accJ_ref[...] = jnp.zeros_like(accJ_ref)
    accD_ref[...] = jnp.zeros_like(accD_ref)
    accC_ref[...] = jnp.zeros_like(accC_ref)

  accJ_ref[...] += jnp.dot(onehot, jk, preferred_element_type=jnp.float32)
  accD_ref[...] += jnp.dot(onehot, delay, preferred_element_type=jnp.float32)
  accC_ref[...] += jnp.sum(onehot, axis=1, keepdims=True)

  @pl.when(i == pl.num_programs(0) - 1)
  def _():
    gden = jnp.maximum(accC_ref[...], 1.0)
    x_class = accD_ref[...] / gden
    x_pool = accJ_ref[...] / gden
    reg_in = jnp.concatenate([other_ref[:, :17], x_class, x_pool], axis=1)
    r = jnp.maximum(jnp.dot(reg_in, Rw1_ref[...],
                            preferred_element_type=jnp.float32)
                    + Rb1_ref[...][None, :], 0.0)
    reg_ref[...] = jnp.dot(r, Rw2_ref[...],
                           preferred_element_type=jnp.float32) \
        + Rb2_ref[...]


def _tc_final(aggP, cnt, h2s, Wl, Wr, bl, g, be, f1, xc, batch3d,
              other_attrs, W1, b1, W2, b2, W3, b3, Rw1, Rb1, Rw2, Rb2):
  grid = N // BT
  return pl.pallas_call(
      _tc_final_body,
      grid=(grid,),
      in_specs=[
          pl.BlockSpec((2, BT, 64), lambda i: (0, i, 0)),
          pl.BlockSpec((2, BT, 16), lambda i: (0, i, 0)),
          pl.BlockSpec((2, BT, 64), lambda i: (0, i, 0)),
          pl.BlockSpec((128, 128), lambda i: (0, 0)),
          pl.BlockSpec((128, 128), lambda i: (0, 0)),
          pl.BlockSpec((128,), lambda i: (0,)),
          pl.BlockSpec((128,), lambda i: (0,)),
          pl.BlockSpec((128,), lambda i: (0,)),
          pl.BlockSpec((2, BT, 64), lambda i: (0, i, 0)),
          pl.BlockSpec((BT, 1), lambda i: (i, 0)),
          pl.BlockSpec((1, 1, BT), lambda i: (i, 0, 0)),
          pl.BlockSpec((G, 18), lambda i: (0, 0)),
          pl.BlockSpec((384, 256), lambda i: (0, 0)),
          pl.BlockSpec((256,), lambda i: (0,)),
          pl.BlockSpec((256, 64), lambda i: (0, 0)),
          pl.BlockSpec((64,), lambda i: (0,)),
          pl.BlockSpec((64, 1), lambda i: (0, 0)),
          pl.BlockSpec((1, 1), lambda i: (0, 0)),
          pl.BlockSpec((402, 32), lambda i: (0, 0)),
          pl.BlockSpec((32,), lambda i: (0,)),
          pl.BlockSpec((32, 1), lambda i: (0, 0)),
          pl.BlockSpec((1, 1), lambda i: (0, 0)),
      ],
      out_specs=[
          pl.BlockSpec((BT, 1), lambda i: (i, 0)),
          pl.BlockSpec((G, 1), lambda i: (0, 0)),
      ],
      out_shape=[
          jax.ShapeDtypeStruct((N, 1), jnp.float32),
          jax.ShapeDtypeStruct((G, 1), jnp.float32),
      ],
      scratch_shapes=[
          pltpu.VMEM((G, 384), jnp.float32),
          pltpu.VMEM((G, 1), jnp.float32),
          pltpu.VMEM((G, 1), jnp.float32),
      ],
  )(aggP, cnt, h2s, Wl, Wr, bl, g, be, f1, xc, batch3d, other_attrs,
    W1, b1, W2, b2, W3, b3, Rw1, Rb1, Rw2, Rb2)


def kernel(x, edge_index, batch, other_attrs, Wl0, bl0, Wr0, g0, be0,
           Wl1, bl1, Wr1, g1, be1, Wl2, bl2, Wr2, g2, be2,
           W1, b1, W2, b2, W3, b3, Rw1, Rb1, Rw2, Rb2):
  src = edge_index[0].astype(jnp.int32)
  dst = edge_index[1].astype(jnp.int32)
  npad = EPAD - E
  src_p = jnp.concatenate([src, jnp.zeros((npad,), jnp.int32)]
                          ).reshape(EPAD // CHUNK, CHUNK)
  dst_p = jnp.concatenate([dst, jnp.full((npad,), N, jnp.int32)]
                          ).reshape(EPAD // CHUNK, CHUNK)
  srcp2 = jnp.stack([src_p, src_p + N])

  xsplit = jnp.stack([x[:, :64], x[:, 64:]])

  def h2d(hs):
    return hs.reshape(2 * N, 64)

  cnt = _sc_cnt(dst_p)
  aggP0 = _sc_seg(h2d(xsplit), srcp2, dst_p)
  h1 = _tc_layer(aggP0, cnt, xsplit, Wl0, Wr0, bl0, g0, be0)
  aggP1 = _sc_seg(h2d(h1), srcp2, dst_p)
  h2 = _tc_layer(aggP1, cnt, h1, Wl1, Wr1, bl1, g1, be1)
  aggP2 = _sc_seg(h2d(h2), srcp2, dst_p)

  xc = x[:, 0:1]
  batch3d = batch.astype(jnp.int32).reshape(N // BT, 1, BT)
  node_output, reg_output = _tc_final(
      aggP2, cnt, h2, Wl2, Wr2, bl2, g2, be2, h1, xc, batch3d, other_attrs,
      W1, b1, W2, b2, W3, b3.reshape(1, 1), Rw1, Rb1, Rw2, Rb2.reshape(1, 1))
  last_attr = other_attrs[:, -1:]
  return (node_output, reg_output, last_attr)
